# Initial kernel scaffold; baseline (speedup 1.0000x reference)
import jax
import jax.numpy as jnp
from jax.experimental import pallas as pl


def kernel(x, edge_index, batch, W1, b1, W2, b2, Wa, a_src, a_dst, ba, Wf1, bf1, Wf2, bf2):
    raise NotImplementedError("write your pallas kernel here")


# speedup vs baseline: 23.3645x; 23.3645x over previous
import functools

import jax
import jax.numpy as jnp
from jax import lax
from jax.experimental import pallas as pl
from jax.experimental.pallas import tpu as pltpu
from jax.experimental.pallas import tpu_sc as plsc

NCORE = 2
NTILE = 16
LANES = 16
C = 128
PAD = 16
RB = 128

_SC_PARAMS = pltpu.CompilerParams(needs_layout_passes=False,
                                  use_tc_tiling_on_sc=False)


def _mesh():
    return plsc.VectorSubcoreMesh(
        core_axis_name="c", subcore_axis_name="s",
        num_cores=NCORE, num_subcores=NTILE)


def _wid():
    return lax.axis_index("s")


def _zero_rows(buf, nrows, ncols):
    zeros16 = jnp.zeros((LANES,), jnp.float32)

    def row(r, _):
        for k in range(ncols // LANES):
            buf[r, pl.ds(k * LANES, LANES)] = zeros16
        return 0

    lax.fori_loop(0, nrows, row, 0)


def _blocks(n):
    nbf = n // RB
    tail = n - nbf * RB
    assert tail % 8 == 0
    return nbf, tail


def _nblk(w, nbf):
    return nbf // NTILE + jnp.where(w < nbf % NTILE, 1, 0)


def _fill_spmem(accum, iobuf, n):
    w = _wid()
    nbf, tail = _blocks(n)
    nb = _nblk(w, nbf)

    def blk(k, _):
        base = (w + k * NTILE) * RB
        pltpu.sync_copy(iobuf, accum.at[pl.ds(base, RB)])
        return 0

    lax.fori_loop(0, nb, blk, 0)
    if tail:
        @pl.when(w == NTILE - 1)
        def _():
            pltpu.sync_copy(iobuf.at[pl.ds(0, tail)],
                            accum.at[pl.ds(nbf * RB, tail)])


def _drain_spmem(accum, iobuf, out_slice_fn, n):
    w = _wid()
    nbf, tail = _blocks(n)
    nb = _nblk(w, nbf)

    def blk(k, _):
        base = (w + k * NTILE) * RB
        pltpu.sync_copy(accum.at[pl.ds(base, RB)], iobuf)
        pltpu.sync_copy(iobuf, out_slice_fn(base, RB))
        return 0

    lax.fori_loop(0, nb, blk, 0)
    if tail:
        @pl.when(w == NTILE - 1)
        def _():
            pltpu.sync_copy(accum.at[pl.ds(nbf * RB, tail)],
                            iobuf.at[pl.ds(0, tail)])
            pltpu.sync_copy(iobuf.at[pl.ds(0, tail)],
                            out_slice_fn(nbf * RB, tail))


def _tile_chunks(per_part, w):
    return per_part // NTILE + jnp.where(w < per_part % NTILE, 1, 0)


def _sc_degree(n, e):
    per_core = (e // C) // NCORE
    mesh = _mesh()

    @functools.partial(
        pl.kernel, mesh=mesh, compiler_params=_SC_PARAMS,
        out_type=jax.ShapeDtypeStruct((NCORE, n, PAD), jnp.float32),
        scratch_types=[
            pltpu.VMEM((C,), jnp.int32),
            pltpu.VMEM((C, PAD), jnp.float32),
            pltpu.VMEM((RB, PAD), jnp.float32),
            pltpu.VMEM_SHARED((n, PAD), jnp.float32),
        ],
    )
    def k(dst_hbm, out_hbm, didx, ones_buf, iobuf, accum):
        c = lax.axis_index("c")
        w = _wid()
        pat = jnp.where(lax.iota(jnp.int32, LANES) == 0, 1.0, 0.0)

        def row(r, _):
            ones_buf[r, pl.ds(0, LANES)] = pat
            return 0

        lax.fori_loop(0, C, row, 0)
        _zero_rows(iobuf, RB, PAD)
        _fill_spmem(accum, iobuf, n)
        plsc.subcore_barrier()

        def chunk(jj, _):
            ebase = (c * per_core + w + jj * NTILE) * C
            pltpu.sync_copy(dst_hbm.at[pl.ds(ebase, C)], didx)
            pltpu.sync_copy(ones_buf, accum.at[didx], add=True)
            return 0

        lax.fori_loop(0, _tile_chunks(per_core, w), chunk, 0)
        plsc.subcore_barrier()
        _drain_spmem(accum, iobuf,
                     lambda base, sz: out_hbm.at[c, pl.ds(base, sz)], n)

    return k


def _sc_edge(n, e, h):
    per_core = (e // C) // NCORE
    mesh = _mesh()

    @functools.partial(
        pl.kernel, mesh=mesh, compiler_params=_SC_PARAMS,
        out_type=jax.ShapeDtypeStruct((NCORE, n, h), jnp.float32),
        scratch_types=[
            pltpu.VMEM((C,), jnp.int32),
            pltpu.VMEM((C,), jnp.int32),
            pltpu.VMEM((C, h), jnp.float32),
            pltpu.VMEM((RB, h), jnp.float32),
            pltpu.VMEM_SHARED((n, h), jnp.float32),
            pltpu.SemaphoreType.DMA,
        ],
    )
    def k(g_hbm, src_hbm, dst_hbm, out_hbm, sidx, didx, rows, iobuf,
          accum, sem):
        c = lax.axis_index("c")
        w = _wid()
        _zero_rows(iobuf, RB, h)
        _fill_spmem(accum, iobuf, n)
        plsc.subcore_barrier()

        def chunk(jj, _):
            ebase = (c * per_core + w + jj * NTILE) * C
            pltpu.sync_copy(src_hbm.at[pl.ds(ebase, C)], sidx)
            pltpu.sync_copy(dst_hbm.at[pl.ds(ebase, C)], didx)
            pltpu.async_copy(g_hbm.at[sidx], rows, sem).wait()
            pltpu.sync_copy(rows, accum.at[didx], add=True)
            return 0

        lax.fori_loop(0, _tile_chunks(per_core, w), chunk, 0)
        plsc.subcore_barrier()
        _drain_spmem(accum, iobuf,
                     lambda base, sz: out_hbm.at[c, pl.ds(base, sz)], n)

    return k


def _sc_gat1(n, e, heads):
    per_core = (e // C) // NCORE
    mesh = _mesh()
    ncols = 2 * heads

    @functools.partial(
        pl.kernel, mesh=mesh, compiler_params=_SC_PARAMS,
        out_type=(
            jax.ShapeDtypeStruct((NCORE, n, PAD), jnp.float32),
            jax.ShapeDtypeStruct((e, PAD), jnp.float32),
        ),
        scratch_types=[
            pltpu.VMEM((n * ncols,), jnp.float32),
            pltpu.VMEM((C,), jnp.int32),
            pltpu.VMEM((C,), jnp.int32),
            pltpu.VMEM((C, PAD), jnp.float32),
            pltpu.VMEM((RB, PAD), jnp.float32),
            pltpu.VMEM_SHARED((n, PAD), jnp.float32),
        ],
    )
    def k(ad_hbm, src_hbm, dst_hbm, s_out, ex_out, ad_tab, sidx, didx,
          exbuf, iobuf, accum):
        c = lax.axis_index("c")
        w = _wid()
        pltpu.sync_copy(ad_hbm, ad_tab)
        _zero_rows(iobuf, RB, PAD)
        _fill_spmem(accum, iobuf, n)
        plsc.subcore_barrier()

        iota16 = lax.iota(jnp.int32, LANES)
        lane_ok = iota16 < heads
        maxi = n * ncols - 1

        def chunk(jj, _):
            ebase = (c * per_core + w + jj * NTILE) * C
            pltpu.sync_copy(src_hbm.at[pl.ds(ebase, C)], sidx)
            pltpu.sync_copy(dst_hbm.at[pl.ds(ebase, C)], didx)
            for g in range(C // LANES):
                sv = sidx[pl.ds(g * LANES, LANES)]
                dv = didx[pl.ds(g * LANES, LANES)]
                for j in range(LANES):
                    si = sv[j] * ncols + iota16
                    di = dv[j] * ncols + heads + iota16
                    a = plsc.load_gather(ad_tab, [jnp.minimum(si, maxi)])
                    b = plsc.load_gather(ad_tab, [jnp.minimum(di, maxi)])
                    ee = a + b
                    ee = jnp.where(ee > 0, ee, 0.2 * ee)
                    ex = jnp.where(lane_ok, jnp.exp(ee), 0.0)
                    exbuf[g * LANES + j, pl.ds(0, LANES)] = ex
            pltpu.sync_copy(exbuf, accum.at[didx], add=True)
            pltpu.sync_copy(exbuf, ex_out.at[pl.ds(ebase, C)])
            return 0

        lax.fori_loop(0, _tile_chunks(per_core, w), chunk, 0)
        plsc.subcore_barrier()
        _drain_spmem(accum, iobuf,
                     lambda base, sz: s_out.at[c, pl.ds(base, sz)], n)

    return k


def _sc_gat2(n, e, heads, hdim):
    per_core = (e // C) // NCORE
    mesh = _mesh()

    @functools.partial(
        pl.kernel, mesh=mesh, compiler_params=_SC_PARAMS,
        out_type=jax.ShapeDtypeStruct((heads, NCORE, n, hdim),
                                      jnp.float32),
        scratch_types=[
            pltpu.VMEM((n * heads,), jnp.float32),
            pltpu.VMEM((C,), jnp.int32),
            pltpu.VMEM((C,), jnp.int32),
            pltpu.VMEM((C * PAD,), jnp.float32),
            pltpu.VMEM((C, hdim), jnp.float32),
            pltpu.VMEM((C,), jnp.float32),
            pltpu.VMEM((RB, hdim), jnp.float32),
            pltpu.VMEM_SHARED((n, hdim), jnp.float32),
            pltpu.SemaphoreType.DMA,
        ],
    )
    def k(hh0_hbm, hh1_hbm, hh2_hbm, hh3_hbm, sinv_hbm, ex_hbm, src_hbm,
          dst_hbm, out_hbm, sinv_tab, sidx, didx, exch, rows, albuf,
          iobuf, accum, sem):
        c = lax.axis_index("c")
        w = _wid()
        pltpu.sync_copy(sinv_hbm, sinv_tab)
        iota16 = lax.iota(jnp.int32, LANES)
        nj = _tile_chunks(per_core, w)

        for hd, hh_hbm in enumerate([hh0_hbm, hh1_hbm, hh2_hbm, hh3_hbm]):
            plsc.subcore_barrier()
            _zero_rows(iobuf, RB, hdim)
            _fill_spmem(accum, iobuf, n)
            plsc.subcore_barrier()

            def chunk(jj, _):
                ebase = (c * per_core + w + jj * NTILE) * C
                pltpu.sync_copy(src_hbm.at[pl.ds(ebase, C)], sidx)
                pltpu.sync_copy(dst_hbm.at[pl.ds(ebase, C)], didx)
                pltpu.sync_copy(ex_hbm.at[pl.ds(ebase * PAD, C * PAD)],
                                exch)
                pltpu.async_copy(hh_hbm.at[sidx], rows, sem).wait()
                for g in range(C // LANES):
                    dv = didx[pl.ds(g * LANES, LANES)]
                    eidx = iota16 * PAD + g * LANES * PAD + hd
                    exv = plsc.load_gather(exch, [eidx])
                    siv = plsc.load_gather(sinv_tab,
                                           [dv * heads + hd])
                    albuf[pl.ds(g * LANES, LANES)] = exv * siv

                def scale(g2, _):
                    va = albuf[pl.ds(g2 * LANES, LANES)]
                    for j in range(LANES):
                        i = g2 * LANES + j
                        aA = va[j]
                        for kk in range(hdim // LANES):
                            v = rows[i, pl.ds(kk * LANES, LANES)]
                            rows[i, pl.ds(kk * LANES, LANES)] = v * aA
                    return 0

                lax.fori_loop(0, C // LANES, scale, 0)
                pltpu.sync_copy(rows, accum.at[didx], add=True)
                return 0

            lax.fori_loop(0, nj, chunk, 0)
            plsc.subcore_barrier()
            _drain_spmem(
                accum, iobuf,
                lambda base, sz: out_hbm.at[hd, c, pl.ds(base, sz)], n)

    return k


def _tc_prep(x, w1, deg_p):
    def body(x_ref, w1_ref, degp_ref, g_ref, hw_ref, dinv_ref):
        deg = degp_ref[0][:, 0:1] + degp_ref[1][:, 0:1] + 1.0
        dinv = lax.rsqrt(deg)
        hw = jnp.dot(x_ref[...], w1_ref[...],
                     preferred_element_type=jnp.float32)
        hw_ref[...] = hw
        g_ref[...] = dinv * hw
        dinv_ref[...] = dinv

    n = x.shape[0]
    h = w1.shape[1]
    return pl.pallas_call(
        body,
        compiler_params=pltpu.CompilerParams(
            vmem_limit_bytes=100 * 1024 * 1024),
        out_shape=(
            jax.ShapeDtypeStruct((n, h), jnp.float32),
            jax.ShapeDtypeStruct((n, h), jnp.float32),
            jax.ShapeDtypeStruct((n, 1), jnp.float32),
        ),
    )(x, w1, deg_p)


def _tc_layer(p, hw, dinv, b, w_next):
    def body(p_ref, hw_ref, dinv_ref, b_ref, w_ref, g_ref, hw2_ref):
        dinv = dinv_ref[...]
        hcur = dinv * (p_ref[0] + p_ref[1]) + dinv * dinv * hw_ref[...]
        hcur = jnp.maximum(hcur + b_ref[...], 0.0)
        hw2 = jnp.dot(hcur, w_ref[...], preferred_element_type=jnp.float32)
        hw2_ref[...] = hw2
        g_ref[...] = dinv * hw2

    n = hw.shape[0]
    h2 = w_next.shape[1]
    return pl.pallas_call(
        body,
        compiler_params=pltpu.CompilerParams(
            vmem_limit_bytes=100 * 1024 * 1024),
        out_shape=(
            jax.ShapeDtypeStruct((n, h2), jnp.float32),
            jax.ShapeDtypeStruct((n, h2), jnp.float32),
        ),
    )(p, hw, dinv, b, w_next)


def _tc_gat_prep(p, hw, dinv, b, wa, aa, heads, hdim):
    def body(p_ref, hw_ref, dinv_ref, b_ref, wa_ref, aa_ref,
             hh0_ref, hh1_ref, hh2_ref, hh3_ref, ad_ref, exs_ref):
        dinv = dinv_ref[...]
        hcur = dinv * (p_ref[0] + p_ref[1]) + dinv * dinv * hw_ref[...]
        hcur = jnp.maximum(hcur + b_ref[...], 0.0)
        hh = jnp.dot(hcur, wa_ref[...], preferred_element_type=jnp.float32)
        hh0_ref[...] = hh[:, 0 * hdim:1 * hdim]
        hh1_ref[...] = hh[:, 1 * hdim:2 * hdim]
        hh2_ref[...] = hh[:, 2 * hdim:3 * hdim]
        hh3_ref[...] = hh[:, 3 * hdim:4 * hdim]
        ad = jnp.dot(hh, aa_ref[...], preferred_element_type=jnp.float32)
        ad_ref[...] = ad
        es = ad[:, :heads] + ad[:, heads:]
        es = jnp.where(es > 0, es, 0.2 * es)
        exs_ref[...] = jnp.exp(es)

    n = hw.shape[0]
    return pl.pallas_call(
        body,
        compiler_params=pltpu.CompilerParams(
            vmem_limit_bytes=100 * 1024 * 1024),
        out_shape=(
            jax.ShapeDtypeStruct((n, hdim), jnp.float32),
            jax.ShapeDtypeStruct((n, hdim), jnp.float32),
            jax.ShapeDtypeStruct((n, hdim), jnp.float32),
            jax.ShapeDtypeStruct((n, hdim), jnp.float32),
            jax.ShapeDtypeStruct((n, 2 * heads), jnp.float32),
            jax.ShapeDtypeStruct((n, heads), jnp.float32),
        ),
    )(p, hw, dinv, b, wa, aa)


def _tc_gat_mid(s_p, exs, hh0, hh1, hh2, hh3, heads, hdim):
    def body(sp_ref, exs_ref, hh0_ref, hh1_ref, hh2_ref, hh3_ref,
             sinv_ref, sg_ref):
        s = sp_ref[0][:, :heads] + sp_ref[1][:, :heads] + exs_ref[...]
        sinv = 1.0 / jnp.maximum(s, 1e-16)
        sinv_ref[...] = sinv
        w0 = exs_ref[...] * sinv
        sg_ref[...] = jnp.concatenate(
            [w0[:, 0:1] * hh0_ref[...], w0[:, 1:2] * hh1_ref[...],
             w0[:, 2:3] * hh2_ref[...], w0[:, 3:4] * hh3_ref[...]],
            axis=1)

    n = exs.shape[0]
    nb = 10
    bn = n // nb
    pad16 = s_p.shape[2]
    hhspec = pl.BlockSpec((bn, hdim), lambda i: (i, 0))
    return pl.pallas_call(
        body,
        grid=(nb,),
        in_specs=[
            pl.BlockSpec((2, bn, pad16), lambda i: (0, i, 0)),
            pl.BlockSpec((bn, heads), lambda i: (i, 0)),
            hhspec, hhspec, hhspec, hhspec,
        ],
        out_specs=(
            pl.BlockSpec((bn, heads), lambda i: (i, 0)),
            pl.BlockSpec((bn, heads * hdim), lambda i: (i, 0)),
        ),
        out_shape=(
            jax.ShapeDtypeStruct((n, heads), jnp.float32),
            jax.ShapeDtypeStruct((n, heads * hdim), jnp.float32),
        ),
    )(s_p, exs, hh0, hh1, hh2, hh3)


def _tc_final(gat_p, sg, ba, batch2d, wf1, bf1, wf2, bf2, ng, heads,
              hdim):
    n = sg.shape[0]
    nb = 10
    bn = n // nb
    nc = wf2.shape[1]

    def body(g_ref, sg_ref, ba_ref, b_ref, wf1_ref, bf1_ref, wf2_ref,
             bf2_ref, out_ref, summ_acc, cnt_acc):
        i = pl.program_id(0)
        parts = [g_ref[hd][0] + g_ref[hd][1] for hd in range(heads)]
        gat = jnp.concatenate(parts, axis=1) + sg_ref[...] + ba_ref[...]
        gid = jax.lax.broadcasted_iota(jnp.int32, (bn, ng), 1)
        oh = (b_ref[...] == gid).astype(jnp.float32)
        summ = lax.dot_general(oh, gat, (((0,), (0,)), ((), ())),
                               preferred_element_type=jnp.float32)
        cnt = lax.dot_general(oh, jnp.ones((bn, 1), jnp.float32),
                              (((0,), (0,)), ((), ())),
                              preferred_element_type=jnp.float32)

        @pl.when(i == 0)
        def _():
            summ_acc[...] = jnp.zeros_like(summ_acc)
            cnt_acc[...] = jnp.zeros_like(cnt_acc)

        summ_acc[...] += summ
        cnt_acc[...] += cnt

        @pl.when(i == nb - 1)
        def _():
            pooled = summ_acc[...] / jnp.maximum(cnt_acc[...], 1.0)
            o = jnp.maximum(
                jnp.dot(pooled, wf1_ref[...],
                        preferred_element_type=jnp.float32)
                + bf1_ref[...], 0.0)
            out_ref[...] = jnp.dot(
                o, wf2_ref[...], preferred_element_type=jnp.float32) \
                + bf2_ref[...]

    full = lambda *shape: pl.BlockSpec(shape, lambda i: tuple(
        0 for _ in shape))
    return pl.pallas_call(
        body,
        grid=(nb,),
        in_specs=[
            pl.BlockSpec((heads, 2, bn, hdim), lambda i: (0, 0, i, 0)),
            pl.BlockSpec((bn, heads * hdim), lambda i: (i, 0)),
            full(1, heads * hdim),
            pl.BlockSpec((bn, 1), lambda i: (i, 0)),
            full(*wf1.shape),
            full(*bf1.shape),
            full(*wf2.shape),
            full(*bf2.shape),
        ],
        out_specs=pl.BlockSpec((ng, nc), lambda i: (0, 0)),
        scratch_shapes=[
            pltpu.VMEM((ng, heads * hdim), jnp.float32),
            pltpu.VMEM((ng, 1), jnp.float32),
        ],
        out_shape=jax.ShapeDtypeStruct((ng, nc), jnp.float32),
    )(gat_p, sg, ba, batch2d, wf1, bf1, wf2, bf2)


def kernel(x, edge_index, batch, W1, b1, W2, b2, Wa, a_src, a_dst, ba,
           Wf1, bf1, Wf2, bf2):
    n, f = x.shape
    e = edge_index.shape[1]
    heads, hdim = a_src.shape
    ng = 64
    h = W1.shape[1]

    src = edge_index[0]
    dst = edge_index[1]
    batch2d = batch.reshape(n, 1)
    b1r = b1.reshape(1, h)
    b2r = b2.reshape(1, h)
    bar = ba.reshape(1, heads * hdim)
    bf1r = bf1.reshape(1, -1)
    bf2r = bf2.reshape(1, -1)
    eye = jnp.eye(heads, dtype=jnp.float32)
    asrc_m = (a_src[:, :, None] * eye[:, None, :]).reshape(heads * hdim,
                                                           heads)
    adst_m = (a_dst[:, :, None] * eye[:, None, :]).reshape(heads * hdim,
                                                           heads)
    aa = jnp.concatenate([asrc_m, adst_m], axis=1)

    deg_p = _sc_degree(n, e)(dst)
    g1, hw1, dinv = _tc_prep(x, W1, deg_p)
    p1 = _sc_edge(n, e, h)(g1, src, dst)
    g2, hw2 = _tc_layer(p1, hw1, dinv, b1r, W2)
    p2 = _sc_edge(n, e, h)(g2, src, dst)
    hh0, hh1, hh2, hh3, ad, exs = _tc_gat_prep(p2, hw2, dinv, b2r, Wa,
                                               aa, heads, hdim)
    s_p, exv = _sc_gat1(n, e, heads)(ad.reshape(-1), src, dst)
    sinv, sg = _tc_gat_mid(s_p, exs, hh0, hh1, hh2, hh3, heads, hdim)
    gat_p = _sc_gat2(n, e, heads, hdim)(hh0, hh1, hh2, hh3,
                                        sinv.reshape(-1),
                                        exv.reshape(-1), src, dst)
    return _tc_final(gat_p, sg, bar, batch2d, Wf1, bf1r, Wf2, bf2r, ng,
                     heads, hdim)


# speedup vs baseline: 24.9332x; 1.0671x over previous
import functools

import jax
import jax.numpy as jnp
from jax import lax
from jax.experimental import pallas as pl
from jax.experimental.pallas import tpu as pltpu
from jax.experimental.pallas import tpu_sc as plsc

NCORE = 2
NTILE = 16
LANES = 16
C = 128
PAD = 16
RB = 128

_SC_PARAMS = pltpu.CompilerParams(needs_layout_passes=False,
                                  use_tc_tiling_on_sc=False)


def _mesh():
    return plsc.VectorSubcoreMesh(
        core_axis_name="c", subcore_axis_name="s",
        num_cores=NCORE, num_subcores=NTILE)


def _wid():
    return lax.axis_index("s")


def _zero_rows(buf, nrows, ncols):
    zeros16 = jnp.zeros((LANES,), jnp.float32)

    def row(r, _):
        for k in range(ncols // LANES):
            buf[r, pl.ds(k * LANES, LANES)] = zeros16
        return 0

    lax.fori_loop(0, nrows, row, 0)


def _blocks(n):
    nbf = n // RB
    tail = n - nbf * RB
    assert tail % 8 == 0
    return nbf, tail


def _nblk(w, nbf):
    return nbf // NTILE + jnp.where(w < nbf % NTILE, 1, 0)


def _fill_spmem(accum, iobuf, n):
    w = _wid()
    nbf, tail = _blocks(n)
    nb = _nblk(w, nbf)

    def blk(k, _):
        base = (w + k * NTILE) * RB
        pltpu.sync_copy(iobuf, accum.at[pl.ds(base, RB)])
        return 0

    lax.fori_loop(0, nb, blk, 0)
    if tail:
        @pl.when(w == NTILE - 1)
        def _():
            pltpu.sync_copy(iobuf.at[pl.ds(0, tail)],
                            accum.at[pl.ds(nbf * RB, tail)])


def _drain_spmem(accum, iobuf, out_slice_fn, n):
    w = _wid()
    nbf, tail = _blocks(n)
    nb = _nblk(w, nbf)

    def blk(k, _):
        base = (w + k * NTILE) * RB
        pltpu.sync_copy(accum.at[pl.ds(base, RB)], iobuf)
        pltpu.sync_copy(iobuf, out_slice_fn(base, RB))
        return 0

    lax.fori_loop(0, nb, blk, 0)
    if tail:
        @pl.when(w == NTILE - 1)
        def _():
            pltpu.sync_copy(accum.at[pl.ds(nbf * RB, tail)],
                            iobuf.at[pl.ds(0, tail)])
            pltpu.sync_copy(iobuf.at[pl.ds(0, tail)],
                            out_slice_fn(nbf * RB, tail))


def _tile_chunks(per_part, w):
    return per_part // NTILE + jnp.where(w < per_part % NTILE, 1, 0)


def _sc_degree(n, e):
    per_core = (e // C) // NCORE
    mesh = _mesh()

    @functools.partial(
        pl.kernel, mesh=mesh, compiler_params=_SC_PARAMS,
        out_type=jax.ShapeDtypeStruct((NCORE, n, PAD), jnp.float32),
        scratch_types=[
            pltpu.VMEM((C,), jnp.int32),
            pltpu.VMEM((C, PAD), jnp.float32),
            pltpu.VMEM((RB, PAD), jnp.float32),
            pltpu.VMEM_SHARED((n, PAD), jnp.float32),
        ],
    )
    def k(dst_hbm, out_hbm, didx, ones_buf, iobuf, accum):
        c = lax.axis_index("c")
        w = _wid()
        pat = jnp.where(lax.iota(jnp.int32, LANES) == 0, 1.0, 0.0)

        def row(r, _):
            ones_buf[r, pl.ds(0, LANES)] = pat
            return 0

        lax.fori_loop(0, C, row, 0)
        _zero_rows(iobuf, RB, PAD)
        _fill_spmem(accum, iobuf, n)
        plsc.subcore_barrier()

        def chunk(jj, _):
            ebase = (c * per_core + w + jj * NTILE) * C
            pltpu.sync_copy(dst_hbm.at[pl.ds(ebase, C)], didx)
            pltpu.sync_copy(ones_buf, accum.at[didx], add=True)
            return 0

        lax.fori_loop(0, _tile_chunks(per_core, w), chunk, 0)
        plsc.subcore_barrier()
        _drain_spmem(accum, iobuf,
                     lambda base, sz: out_hbm.at[c, pl.ds(base, sz)], n)

    return k


def _sc_edge(n, e, h):
    per_core = (e // C) // NCORE
    mesh = _mesh()

    @functools.partial(
        pl.kernel, mesh=mesh, compiler_params=_SC_PARAMS,
        out_type=jax.ShapeDtypeStruct((NCORE, n, h), jnp.float32),
        scratch_types=[
            pltpu.VMEM((2, C), jnp.int32),
            pltpu.VMEM((2, C), jnp.int32),
            pltpu.VMEM((2, C, h), jnp.float32),
            pltpu.VMEM((RB, h), jnp.float32),
            pltpu.VMEM_SHARED((n, h), jnp.float32),
            pltpu.SemaphoreType.DMA,
            pltpu.SemaphoreType.DMA,
        ],
    )
    def k(g_hbm, src_hbm, dst_hbm, out_hbm, sidx, didx, rows, iobuf,
          accum, sem0, sem1):
        c = lax.axis_index("c")
        w = _wid()
        _zero_rows(iobuf, RB, h)
        _fill_spmem(accum, iobuf, n)
        plsc.subcore_barrier()

        nj = _tile_chunks(per_core, w)
        sems = (sem0, sem1)

        def issue(jj, par):
            ebase = (c * per_core + w + jj * NTILE) * C
            pltpu.sync_copy(src_hbm.at[pl.ds(ebase, C)], sidx.at[par])
            pltpu.sync_copy(dst_hbm.at[pl.ds(ebase, C)], didx.at[par])
            pltpu.async_copy(g_hbm.at[sidx.at[par]], rows.at[par],
                             sems[par])

        def process(jj, par):
            pltpu.make_async_copy(g_hbm.at[sidx.at[par]], rows.at[par],
                                  sems[par]).wait()
            pltpu.sync_copy(rows.at[par], accum.at[didx.at[par]],
                            add=True)

        @pl.when(nj > 0)
        def _():
            issue(0, 0)

        def body(jj, _):
            for par in (0, 1):
                @pl.when(jj % 2 == par)
                def _():
                    @pl.when(jj + 1 < nj)
                    def _():
                        issue(jj + 1, 1 - par)

                    process(jj, par)
            return 0

        lax.fori_loop(0, nj, body, 0)
        plsc.subcore_barrier()
        _drain_spmem(accum, iobuf,
                     lambda base, sz: out_hbm.at[c, pl.ds(base, sz)], n)

    return k


def _sc_gat1(n, e, heads):
    per_core = (e // C) // NCORE
    mesh = _mesh()
    ncols = 2 * heads

    @functools.partial(
        pl.kernel, mesh=mesh, compiler_params=_SC_PARAMS,
        out_type=(
            jax.ShapeDtypeStruct((NCORE, n, PAD), jnp.float32),
            jax.ShapeDtypeStruct((e, PAD), jnp.float32),
        ),
        scratch_types=[
            pltpu.VMEM((n * ncols,), jnp.float32),
            pltpu.VMEM((C,), jnp.int32),
            pltpu.VMEM((C,), jnp.int32),
            pltpu.VMEM((C, PAD), jnp.float32),
            pltpu.VMEM((RB, PAD), jnp.float32),
            pltpu.VMEM_SHARED((n, PAD), jnp.float32),
        ],
    )
    def k(ad_hbm, src_hbm, dst_hbm, s_out, ex_out, ad_tab, sidx, didx,
          exbuf, iobuf, accum):
        c = lax.axis_index("c")
        w = _wid()
        pltpu.sync_copy(ad_hbm, ad_tab)
        _zero_rows(iobuf, RB, PAD)
        _fill_spmem(accum, iobuf, n)
        plsc.subcore_barrier()

        iota16 = lax.iota(jnp.int32, LANES)
        lane_ok = iota16 < heads
        maxi = n * ncols - 1

        def chunk(jj, _):
            ebase = (c * per_core + w + jj * NTILE) * C
            pltpu.sync_copy(src_hbm.at[pl.ds(ebase, C)], sidx)
            pltpu.sync_copy(dst_hbm.at[pl.ds(ebase, C)], didx)
            for g in range(C // LANES):
                sv = sidx[pl.ds(g * LANES, LANES)]
                dv = didx[pl.ds(g * LANES, LANES)]
                for j in range(LANES):
                    si = sv[j] * ncols + iota16
                    di = dv[j] * ncols + heads + iota16
                    a = plsc.load_gather(ad_tab, [jnp.minimum(si, maxi)])
                    b = plsc.load_gather(ad_tab, [jnp.minimum(di, maxi)])
                    ee = a + b
                    ee = jnp.where(ee > 0, ee, 0.2 * ee)
                    ex = jnp.where(lane_ok, jnp.exp(ee), 0.0)
                    exbuf[g * LANES + j, pl.ds(0, LANES)] = ex
            pltpu.sync_copy(exbuf, accum.at[didx], add=True)
            pltpu.sync_copy(exbuf, ex_out.at[pl.ds(ebase, C)])
            return 0

        lax.fori_loop(0, _tile_chunks(per_core, w), chunk, 0)
        plsc.subcore_barrier()
        _drain_spmem(accum, iobuf,
                     lambda base, sz: s_out.at[c, pl.ds(base, sz)], n)

    return k


def _sc_gat2(n, e, heads, hdim):
    per_core = (e // C) // NCORE
    mesh = _mesh()

    @functools.partial(
        pl.kernel, mesh=mesh, compiler_params=_SC_PARAMS,
        out_type=jax.ShapeDtypeStruct((heads, NCORE, n, hdim),
                                      jnp.float32),
        scratch_types=[
            pltpu.VMEM((n * heads,), jnp.float32),
            pltpu.VMEM((C,), jnp.int32),
            pltpu.VMEM((C,), jnp.int32),
            pltpu.VMEM((C * PAD,), jnp.float32),
            pltpu.VMEM((C, hdim), jnp.float32),
            pltpu.VMEM((C,), jnp.float32),
            pltpu.VMEM((RB, hdim), jnp.float32),
            pltpu.VMEM_SHARED((n, hdim), jnp.float32),
            pltpu.SemaphoreType.DMA,
        ],
    )
    def k(hh0_hbm, hh1_hbm, hh2_hbm, hh3_hbm, sinv_hbm, ex_hbm, src_hbm,
          dst_hbm, out_hbm, sinv_tab, sidx, didx, exch, rows, albuf,
          iobuf, accum, sem):
        c = lax.axis_index("c")
        w = _wid()
        pltpu.sync_copy(sinv_hbm, sinv_tab)
        iota16 = lax.iota(jnp.int32, LANES)
        nj = _tile_chunks(per_core, w)

        for hd, hh_hbm in enumerate([hh0_hbm, hh1_hbm, hh2_hbm, hh3_hbm]):
            plsc.subcore_barrier()
            _zero_rows(iobuf, RB, hdim)
            _fill_spmem(accum, iobuf, n)
            plsc.subcore_barrier()

            def chunk(jj, _):
                ebase = (c * per_core + w + jj * NTILE) * C
                pltpu.sync_copy(src_hbm.at[pl.ds(ebase, C)], sidx)
                pltpu.sync_copy(dst_hbm.at[pl.ds(ebase, C)], didx)
                pltpu.sync_copy(ex_hbm.at[pl.ds(ebase * PAD, C * PAD)],
                                exch)
                pltpu.async_copy(hh_hbm.at[sidx], rows, sem).wait()
                for g in range(C // LANES):
                    dv = didx[pl.ds(g * LANES, LANES)]
                    eidx = iota16 * PAD + g * LANES * PAD + hd
                    exv = plsc.load_gather(exch, [eidx])
                    siv = plsc.load_gather(sinv_tab,
                                           [dv * heads + hd])
                    albuf[pl.ds(g * LANES, LANES)] = exv * siv

                def scale(g2, _):
                    va = albuf[pl.ds(g2 * LANES, LANES)]
                    for j in range(LANES):
                        i = g2 * LANES + j
                        aA = va[j]
                        for kk in range(hdim // LANES):
                            v = rows[i, pl.ds(kk * LANES, LANES)]
                            rows[i, pl.ds(kk * LANES, LANES)] = v * aA
                    return 0

                lax.fori_loop(0, C // LANES, scale, 0)
                pltpu.sync_copy(rows, accum.at[didx], add=True)
                return 0

            lax.fori_loop(0, nj, chunk, 0)
            plsc.subcore_barrier()
            _drain_spmem(
                accum, iobuf,
                lambda base, sz: out_hbm.at[hd, c, pl.ds(base, sz)], n)

    return k


def _tc_prep(x, w1, deg_p):
    def body(x_ref, w1_ref, degp_ref, g_ref, hw_ref, dinv_ref):
        deg = degp_ref[0][:, 0:1] + degp_ref[1][:, 0:1] + 1.0
        dinv = lax.rsqrt(deg)
        hw = jnp.dot(x_ref[...], w1_ref[...],
                     preferred_element_type=jnp.float32)
        hw_ref[...] = hw
        g_ref[...] = dinv * hw
        dinv_ref[...] = dinv

    n = x.shape[0]
    h = w1.shape[1]
    return pl.pallas_call(
        body,
        compiler_params=pltpu.CompilerParams(
            vmem_limit_bytes=100 * 1024 * 1024),
        out_shape=(
            jax.ShapeDtypeStruct((n, h), jnp.float32),
            jax.ShapeDtypeStruct((n, h), jnp.float32),
            jax.ShapeDtypeStruct((n, 1), jnp.float32),
        ),
    )(x, w1, deg_p)


def _tc_layer(p, hw, dinv, b, w_next):
    def body(p_ref, hw_ref, dinv_ref, b_ref, w_ref, g_ref, hw2_ref):
        dinv = dinv_ref[...]
        hcur = dinv * (p_ref[0] + p_ref[1]) + dinv * dinv * hw_ref[...]
        hcur = jnp.maximum(hcur + b_ref[...], 0.0)
        hw2 = jnp.dot(hcur, w_ref[...], preferred_element_type=jnp.float32)
        hw2_ref[...] = hw2
        g_ref[...] = dinv * hw2

    n = hw.shape[0]
    h2 = w_next.shape[1]
    return pl.pallas_call(
        body,
        compiler_params=pltpu.CompilerParams(
            vmem_limit_bytes=100 * 1024 * 1024),
        out_shape=(
            jax.ShapeDtypeStruct((n, h2), jnp.float32),
            jax.ShapeDtypeStruct((n, h2), jnp.float32),
        ),
    )(p, hw, dinv, b, w_next)


def _tc_gat_prep(p, hw, dinv, b, wa, aa, heads, hdim):
    def body(p_ref, hw_ref, dinv_ref, b_ref, wa_ref, aa_ref,
             hh0_ref, hh1_ref, hh2_ref, hh3_ref, ad_ref, exs_ref):
        dinv = dinv_ref[...]
        hcur = dinv * (p_ref[0] + p_ref[1]) + dinv * dinv * hw_ref[...]
        hcur = jnp.maximum(hcur + b_ref[...], 0.0)
        hh = jnp.dot(hcur, wa_ref[...], preferred_element_type=jnp.float32)
        hh0_ref[...] = hh[:, 0 * hdim:1 * hdim]
        hh1_ref[...] = hh[:, 1 * hdim:2 * hdim]
        hh2_ref[...] = hh[:, 2 * hdim:3 * hdim]
        hh3_ref[...] = hh[:, 3 * hdim:4 * hdim]
        ad = jnp.dot(hh, aa_ref[...], preferred_element_type=jnp.float32)
        ad_ref[...] = ad
        es = ad[:, :heads] + ad[:, heads:]
        es = jnp.where(es > 0, es, 0.2 * es)
        exs_ref[...] = jnp.exp(es)

    n = hw.shape[0]
    return pl.pallas_call(
        body,
        compiler_params=pltpu.CompilerParams(
            vmem_limit_bytes=100 * 1024 * 1024),
        out_shape=(
            jax.ShapeDtypeStruct((n, hdim), jnp.float32),
            jax.ShapeDtypeStruct((n, hdim), jnp.float32),
            jax.ShapeDtypeStruct((n, hdim), jnp.float32),
            jax.ShapeDtypeStruct((n, hdim), jnp.float32),
            jax.ShapeDtypeStruct((n, 2 * heads), jnp.float32),
            jax.ShapeDtypeStruct((n, heads), jnp.float32),
        ),
    )(p, hw, dinv, b, wa, aa)


def _tc_gat_mid(s_p, exs, hh0, hh1, hh2, hh3, heads, hdim):
    def body(sp_ref, exs_ref, hh0_ref, hh1_ref, hh2_ref, hh3_ref,
             sinv_ref, sg_ref):
        s = sp_ref[0][:, :heads] + sp_ref[1][:, :heads] + exs_ref[...]
        sinv = 1.0 / jnp.maximum(s, 1e-16)
        sinv_ref[...] = sinv
        w0 = exs_ref[...] * sinv
        sg_ref[...] = jnp.concatenate(
            [w0[:, 0:1] * hh0_ref[...], w0[:, 1:2] * hh1_ref[...],
             w0[:, 2:3] * hh2_ref[...], w0[:, 3:4] * hh3_ref[...]],
            axis=1)

    n = exs.shape[0]
    nb = 10
    bn = n // nb
    pad16 = s_p.shape[2]
    hhspec = pl.BlockSpec((bn, hdim), lambda i: (i, 0))
    return pl.pallas_call(
        body,
        grid=(nb,),
        in_specs=[
            pl.BlockSpec((2, bn, pad16), lambda i: (0, i, 0)),
            pl.BlockSpec((bn, heads), lambda i: (i, 0)),
            hhspec, hhspec, hhspec, hhspec,
        ],
        out_specs=(
            pl.BlockSpec((bn, heads), lambda i: (i, 0)),
            pl.BlockSpec((bn, heads * hdim), lambda i: (i, 0)),
        ),
        out_shape=(
            jax.ShapeDtypeStruct((n, heads), jnp.float32),
            jax.ShapeDtypeStruct((n, heads * hdim), jnp.float32),
        ),
    )(s_p, exs, hh0, hh1, hh2, hh3)


def _tc_final(gat_p, sg, ba, batch2d, wf1, bf1, wf2, bf2, ng, heads,
              hdim):
    n = sg.shape[0]
    nb = 10
    bn = n // nb
    nc = wf2.shape[1]

    def body(g_ref, sg_ref, ba_ref, b_ref, wf1_ref, bf1_ref, wf2_ref,
             bf2_ref, out_ref, summ_acc, cnt_acc):
        i = pl.program_id(0)
        parts = [g_ref[hd][0] + g_ref[hd][1] for hd in range(heads)]
        gat = jnp.concatenate(parts, axis=1) + sg_ref[...] + ba_ref[...]
        gid = jax.lax.broadcasted_iota(jnp.int32, (bn, ng), 1)
        oh = (b_ref[...] == gid).astype(jnp.float32)
        summ = lax.dot_general(oh, gat, (((0,), (0,)), ((), ())),
                               preferred_element_type=jnp.float32)
        cnt = lax.dot_general(oh, jnp.ones((bn, 1), jnp.float32),
                              (((0,), (0,)), ((), ())),
                              preferred_element_type=jnp.float32)

        @pl.when(i == 0)
        def _():
            summ_acc[...] = jnp.zeros_like(summ_acc)
            cnt_acc[...] = jnp.zeros_like(cnt_acc)

        summ_acc[...] += summ
        cnt_acc[...] += cnt

        @pl.when(i == nb - 1)
        def _():
            pooled = summ_acc[...] / jnp.maximum(cnt_acc[...], 1.0)
            o = jnp.maximum(
                jnp.dot(pooled, wf1_ref[...],
                        preferred_element_type=jnp.float32)
                + bf1_ref[...], 0.0)
            out_ref[...] = jnp.dot(
                o, wf2_ref[...], preferred_element_type=jnp.float32) \
                + bf2_ref[...]

    full = lambda *shape: pl.BlockSpec(shape, lambda i: tuple(
        0 for _ in shape))
    return pl.pallas_call(
        body,
        grid=(nb,),
        in_specs=[
            pl.BlockSpec((heads, 2, bn, hdim), lambda i: (0, 0, i, 0)),
            pl.BlockSpec((bn, heads * hdim), lambda i: (i, 0)),
            full(1, heads * hdim),
            pl.BlockSpec((bn, 1), lambda i: (i, 0)),
            full(*wf1.shape),
            full(*bf1.shape),
            full(*wf2.shape),
            full(*bf2.shape),
        ],
        out_specs=pl.BlockSpec((ng, nc), lambda i: (0, 0)),
        scratch_shapes=[
            pltpu.VMEM((ng, heads * hdim), jnp.float32),
            pltpu.VMEM((ng, 1), jnp.float32),
        ],
        out_shape=jax.ShapeDtypeStruct((ng, nc), jnp.float32),
    )(gat_p, sg, ba, batch2d, wf1, bf1, wf2, bf2)


def kernel(x, edge_index, batch, W1, b1, W2, b2, Wa, a_src, a_dst, ba,
           Wf1, bf1, Wf2, bf2):
    n, f = x.shape
    e = edge_index.shape[1]
    heads, hdim = a_src.shape
    ng = 64
    h = W1.shape[1]

    src = edge_index[0]
    dst = edge_index[1]
    batch2d = batch.reshape(n, 1)
    b1r = b1.reshape(1, h)
    b2r = b2.reshape(1, h)
    bar = ba.reshape(1, heads * hdim)
    bf1r = bf1.reshape(1, -1)
    bf2r = bf2.reshape(1, -1)
    eye = jnp.eye(heads, dtype=jnp.float32)
    asrc_m = (a_src[:, :, None] * eye[:, None, :]).reshape(heads * hdim,
                                                           heads)
    adst_m = (a_dst[:, :, None] * eye[:, None, :]).reshape(heads * hdim,
                                                           heads)
    aa = jnp.concatenate([asrc_m, adst_m], axis=1)

    deg_p = _sc_degree(n, e)(dst)
    g1, hw1, dinv = _tc_prep(x, W1, deg_p)
    p1 = _sc_edge(n, e, h)(g1, src, dst)
    g2, hw2 = _tc_layer(p1, hw1, dinv, b1r, W2)
    p2 = _sc_edge(n, e, h)(g2, src, dst)
    hh0, hh1, hh2, hh3, ad, exs = _tc_gat_prep(p2, hw2, dinv, b2r, Wa,
                                               aa, heads, hdim)
    s_p, exv = _sc_gat1(n, e, heads)(ad.reshape(-1), src, dst)
    sinv, sg = _tc_gat_mid(s_p, exs, hh0, hh1, hh2, hh3, heads, hdim)
    gat_p = _sc_gat2(n, e, heads, hdim)(hh0, hh1, hh2, hh3,
                                        sinv.reshape(-1),
                                        exv.reshape(-1), src, dst)
    return _tc_final(gat_p, sg, bar, batch2d, Wf1, bf1r, Wf2, bf2r, ng,
                     heads, hdim)


# speedup vs baseline: 29.0063x; 1.1634x over previous
import functools

import jax
import jax.numpy as jnp
from jax import lax
from jax.experimental import pallas as pl
from jax.experimental.pallas import tpu as pltpu
from jax.experimental.pallas import tpu_sc as plsc

NCORE = 2
NTILE = 16
LANES = 16
C = 128
PAD = 16
RB = 128

_SC_PARAMS = pltpu.CompilerParams(needs_layout_passes=False,
                                  use_tc_tiling_on_sc=False)


def _mesh():
    return plsc.VectorSubcoreMesh(
        core_axis_name="c", subcore_axis_name="s",
        num_cores=NCORE, num_subcores=NTILE)


def _wid():
    return lax.axis_index("s")


def _zero_rows(buf, nrows, ncols):
    zeros16 = jnp.zeros((LANES,), jnp.float32)

    def row(r, _):
        for k in range(ncols // LANES):
            buf[r, pl.ds(k * LANES, LANES)] = zeros16
        return 0

    lax.fori_loop(0, nrows, row, 0)


def _blocks(n):
    nbf = n // RB
    tail = n - nbf * RB
    assert tail % 8 == 0
    return nbf, tail


def _nblk(w, nbf):
    return nbf // NTILE + jnp.where(w < nbf % NTILE, 1, 0)


def _fill_spmem(accum, iobuf, n):
    w = _wid()
    nbf, tail = _blocks(n)
    nb = _nblk(w, nbf)

    def blk(k, _):
        base = (w + k * NTILE) * RB
        pltpu.sync_copy(iobuf, accum.at[pl.ds(base, RB)])
        return 0

    lax.fori_loop(0, nb, blk, 0)
    if tail:
        @pl.when(w == NTILE - 1)
        def _():
            pltpu.sync_copy(iobuf.at[pl.ds(0, tail)],
                            accum.at[pl.ds(nbf * RB, tail)])


def _drain_spmem(accum, iobuf, out_slice_fn, n):
    w = _wid()
    nbf, tail = _blocks(n)
    nb = _nblk(w, nbf)

    def blk(k, _):
        base = (w + k * NTILE) * RB
        pltpu.sync_copy(accum.at[pl.ds(base, RB)], iobuf)
        pltpu.sync_copy(iobuf, out_slice_fn(base, RB))
        return 0

    lax.fori_loop(0, nb, blk, 0)
    if tail:
        @pl.when(w == NTILE - 1)
        def _():
            pltpu.sync_copy(accum.at[pl.ds(nbf * RB, tail)],
                            iobuf.at[pl.ds(0, tail)])
            pltpu.sync_copy(iobuf.at[pl.ds(0, tail)],
                            out_slice_fn(nbf * RB, tail))


def _tile_chunks(per_part, w):
    return per_part // NTILE + jnp.where(w < per_part % NTILE, 1, 0)


def _sc_degree(n, e):
    per_core = (e // C) // NCORE
    mesh = _mesh()

    @functools.partial(
        pl.kernel, mesh=mesh, compiler_params=_SC_PARAMS,
        out_type=jax.ShapeDtypeStruct((NCORE, n, PAD), jnp.float32),
        scratch_types=[
            pltpu.VMEM((C,), jnp.int32),
            pltpu.VMEM((C, PAD), jnp.float32),
            pltpu.VMEM((RB, PAD), jnp.float32),
            pltpu.VMEM_SHARED((n, PAD), jnp.float32),
        ],
    )
    def k(dst_hbm, out_hbm, didx, ones_buf, iobuf, accum):
        c = lax.axis_index("c")
        w = _wid()
        pat = jnp.where(lax.iota(jnp.int32, LANES) == 0, 1.0, 0.0)

        def row(r, _):
            ones_buf[r, pl.ds(0, LANES)] = pat
            return 0

        lax.fori_loop(0, C, row, 0)
        _zero_rows(iobuf, RB, PAD)
        _fill_spmem(accum, iobuf, n)
        plsc.subcore_barrier()

        def chunk(jj, _):
            ebase = (c * per_core + w + jj * NTILE) * C
            pltpu.sync_copy(dst_hbm.at[pl.ds(ebase, C)], didx)
            pltpu.sync_copy(ones_buf, accum.at[didx], add=True)
            return 0

        lax.fori_loop(0, _tile_chunks(per_core, w), chunk, 0)
        plsc.subcore_barrier()
        _drain_spmem(accum, iobuf,
                     lambda base, sz: out_hbm.at[c, pl.ds(base, sz)], n)

    return k


def _sc_edge(n, e, h):
    per_core = (e // C) // NCORE
    mesh = _mesh()

    @functools.partial(
        pl.kernel, mesh=mesh, compiler_params=_SC_PARAMS,
        out_type=jax.ShapeDtypeStruct((NCORE, n, h), jnp.float32),
        scratch_types=[
            pltpu.VMEM((2, C), jnp.int32),
            pltpu.VMEM((2, C), jnp.int32),
            pltpu.VMEM((2, C, h), jnp.float32),
            pltpu.VMEM((RB, h), jnp.float32),
            pltpu.VMEM_SHARED((n, h), jnp.float32),
            pltpu.SemaphoreType.DMA,
            pltpu.SemaphoreType.DMA,
        ],
    )
    def k(g_hbm, src_hbm, dst_hbm, out_hbm, sidx, didx, rows, iobuf,
          accum, sem0, sem1):
        c = lax.axis_index("c")
        w = _wid()
        _zero_rows(iobuf, RB, h)
        _fill_spmem(accum, iobuf, n)
        plsc.subcore_barrier()

        nj = _tile_chunks(per_core, w)
        sems = (sem0, sem1)

        def issue(jj, par):
            ebase = (c * per_core + w + jj * NTILE) * C
            pltpu.sync_copy(src_hbm.at[pl.ds(ebase, C)], sidx.at[par])
            pltpu.sync_copy(dst_hbm.at[pl.ds(ebase, C)], didx.at[par])
            pltpu.async_copy(g_hbm.at[sidx.at[par]], rows.at[par],
                             sems[par])

        def process(jj, par):
            pltpu.make_async_copy(g_hbm.at[sidx.at[par]], rows.at[par],
                                  sems[par]).wait()
            pltpu.sync_copy(rows.at[par], accum.at[didx.at[par]],
                            add=True)

        @pl.when(nj > 0)
        def _():
            issue(0, 0)

        def body(jj, _):
            for par in (0, 1):
                @pl.when(jj % 2 == par)
                def _():
                    @pl.when(jj + 1 < nj)
                    def _():
                        issue(jj + 1, 1 - par)

                    process(jj, par)
            return 0

        lax.fori_loop(0, nj, body, 0)
        plsc.subcore_barrier()
        _drain_spmem(accum, iobuf,
                     lambda base, sz: out_hbm.at[c, pl.ds(base, sz)], n)

    return k


def _sc_gat1(n, e, heads):
    per_core = (e // C) // NCORE
    mesh = _mesh()
    ncols = 2 * heads

    @functools.partial(
        pl.kernel, mesh=mesh, compiler_params=_SC_PARAMS,
        out_type=(
            jax.ShapeDtypeStruct((NCORE, n, PAD), jnp.float32),
            jax.ShapeDtypeStruct((e, PAD), jnp.float32),
        ),
        scratch_types=[
            pltpu.VMEM((n * ncols,), jnp.float32),
            pltpu.VMEM((C,), jnp.int32),
            pltpu.VMEM((C,), jnp.int32),
            pltpu.VMEM((C, PAD), jnp.float32),
            pltpu.VMEM((RB, PAD), jnp.float32),
            pltpu.VMEM_SHARED((n, PAD), jnp.float32),
        ],
    )
    def k(ad_hbm, src_hbm, dst_hbm, s_out, ex_out, ad_tab, sidx, didx,
          exbuf, iobuf, accum):
        c = lax.axis_index("c")
        w = _wid()
        pltpu.sync_copy(ad_hbm, ad_tab)
        _zero_rows(iobuf, RB, PAD)
        _fill_spmem(accum, iobuf, n)
        plsc.subcore_barrier()

        iota16 = lax.iota(jnp.int32, LANES)
        lane_ok = iota16 < heads
        maxi = n * ncols - 1

        def chunk(jj, _):
            ebase = (c * per_core + w + jj * NTILE) * C
            pltpu.sync_copy(src_hbm.at[pl.ds(ebase, C)], sidx)
            pltpu.sync_copy(dst_hbm.at[pl.ds(ebase, C)], didx)
            for g in range(C // LANES):
                sv = sidx[pl.ds(g * LANES, LANES)]
                dv = didx[pl.ds(g * LANES, LANES)]
                for j in range(LANES):
                    si = sv[j] * ncols + iota16
                    di = dv[j] * ncols + heads + iota16
                    a = plsc.load_gather(ad_tab, [jnp.minimum(si, maxi)])
                    b = plsc.load_gather(ad_tab, [jnp.minimum(di, maxi)])
                    ee = a + b
                    ee = jnp.where(ee > 0, ee, 0.2 * ee)
                    ex = jnp.where(lane_ok, jnp.exp(ee), 0.0)
                    exbuf[g * LANES + j, pl.ds(0, LANES)] = ex
            pltpu.sync_copy(exbuf, accum.at[didx], add=True)
            pltpu.sync_copy(exbuf, ex_out.at[pl.ds(ebase, C)])
            return 0

        lax.fori_loop(0, _tile_chunks(per_core, w), chunk, 0)
        plsc.subcore_barrier()
        _drain_spmem(accum, iobuf,
                     lambda base, sz: s_out.at[c, pl.ds(base, sz)], n)

    return k


def _sc_gat2(n, e, heads, hdim):
    per_core = (e // C) // NCORE
    mesh = _mesh()

    @functools.partial(
        pl.kernel, mesh=mesh, compiler_params=_SC_PARAMS,
        out_type=jax.ShapeDtypeStruct((heads, NCORE, n, hdim),
                                      jnp.float32),
        scratch_types=[
            pltpu.VMEM((n * heads,), jnp.float32),
            pltpu.VMEM((2, C), jnp.int32),
            pltpu.VMEM((2, C), jnp.int32),
            pltpu.VMEM((2, C * PAD), jnp.float32),
            pltpu.VMEM((2, C, hdim), jnp.float32),
            pltpu.VMEM((C,), jnp.float32),
            pltpu.VMEM((RB, hdim), jnp.float32),
            pltpu.VMEM_SHARED((n, hdim), jnp.float32),
            pltpu.SemaphoreType.DMA,
            pltpu.SemaphoreType.DMA,
        ],
    )
    def k(hh0_hbm, hh1_hbm, hh2_hbm, hh3_hbm, sinv_hbm, ex_hbm, src_hbm,
          dst_hbm, out_hbm, sinv_tab, sidx, didx, exch, rows, albuf,
          iobuf, accum, sem0, sem1):
        c = lax.axis_index("c")
        w = _wid()
        pltpu.sync_copy(sinv_hbm, sinv_tab)
        iota16 = lax.iota(jnp.int32, LANES)
        nj = _tile_chunks(per_core, w)
        sems = (sem0, sem1)

        for hd, hh_hbm in enumerate([hh0_hbm, hh1_hbm, hh2_hbm, hh3_hbm]):
            plsc.subcore_barrier()
            _zero_rows(iobuf, RB, hdim)
            _fill_spmem(accum, iobuf, n)
            plsc.subcore_barrier()

            def issue(jj, par, hh_hbm=hh_hbm):
                ebase = (c * per_core + w + jj * NTILE) * C
                pltpu.sync_copy(src_hbm.at[pl.ds(ebase, C)],
                                sidx.at[par])
                pltpu.sync_copy(dst_hbm.at[pl.ds(ebase, C)],
                                didx.at[par])
                pltpu.sync_copy(ex_hbm.at[pl.ds(ebase * PAD, C * PAD)],
                                exch.at[par])
                pltpu.async_copy(hh_hbm.at[sidx.at[par]], rows.at[par],
                                 sems[par])

            def process(jj, par, hd=hd, hh_hbm=hh_hbm):
                pltpu.make_async_copy(hh_hbm.at[sidx.at[par]],
                                      rows.at[par], sems[par]).wait()
                for g in range(C // LANES):
                    dv = didx[par, pl.ds(g * LANES, LANES)]
                    eidx = iota16 * PAD + g * LANES * PAD + hd
                    exv = plsc.load_gather(exch.at[par], [eidx])
                    siv = plsc.load_gather(sinv_tab,
                                           [dv * heads + hd])
                    albuf[pl.ds(g * LANES, LANES)] = exv * siv

                def scale(g2, _):
                    va = albuf[pl.ds(g2 * LANES, LANES)]
                    for j in range(LANES):
                        i = g2 * LANES + j
                        aA = va[j]
                        for kk in range(hdim // LANES):
                            v = rows[par, i, pl.ds(kk * LANES, LANES)]
                            rows[par, i, pl.ds(kk * LANES, LANES)] = \
                                v * aA
                    return 0

                lax.fori_loop(0, C // LANES, scale, 0)
                pltpu.sync_copy(rows.at[par], accum.at[didx.at[par]],
                                add=True)

            @pl.when(nj > 0)
            def _():
                issue(0, 0)

            def body(jj, _):
                for par in (0, 1):
                    @pl.when(jj % 2 == par)
                    def _():
                        @pl.when(jj + 1 < nj)
                        def _():
                            issue(jj + 1, 1 - par)

                        process(jj, par)
                return 0

            lax.fori_loop(0, nj, body, 0)
            plsc.subcore_barrier()
            _drain_spmem(
                accum, iobuf,
                lambda base, sz: out_hbm.at[hd, c, pl.ds(base, sz)], n)

    return k


def _tc_prep(x, w1, deg_p):
    def body(x_ref, w1_ref, degp_ref, g_ref, hw_ref, dinv_ref):
        deg = degp_ref[0][:, 0:1] + degp_ref[1][:, 0:1] + 1.0
        dinv = lax.rsqrt(deg)
        hw = jnp.dot(x_ref[...], w1_ref[...],
                     preferred_element_type=jnp.float32)
        hw_ref[...] = hw
        g_ref[...] = dinv * hw
        dinv_ref[...] = dinv

    n = x.shape[0]
    h = w1.shape[1]
    return pl.pallas_call(
        body,
        compiler_params=pltpu.CompilerParams(
            vmem_limit_bytes=100 * 1024 * 1024),
        out_shape=(
            jax.ShapeDtypeStruct((n, h), jnp.float32),
            jax.ShapeDtypeStruct((n, h), jnp.float32),
            jax.ShapeDtypeStruct((n, 1), jnp.float32),
        ),
    )(x, w1, deg_p)


def _tc_layer(p, hw, dinv, b, w_next):
    def body(p_ref, hw_ref, dinv_ref, b_ref, w_ref, g_ref, hw2_ref):
        dinv = dinv_ref[...]
        hcur = dinv * (p_ref[0] + p_ref[1]) + dinv * dinv * hw_ref[...]
        hcur = jnp.maximum(hcur + b_ref[...], 0.0)
        hw2 = jnp.dot(hcur, w_ref[...], preferred_element_type=jnp.float32)
        hw2_ref[...] = hw2
        g_ref[...] = dinv * hw2

    n = hw.shape[0]
    h2 = w_next.shape[1]
    return pl.pallas_call(
        body,
        compiler_params=pltpu.CompilerParams(
            vmem_limit_bytes=100 * 1024 * 1024),
        out_shape=(
            jax.ShapeDtypeStruct((n, h2), jnp.float32),
            jax.ShapeDtypeStruct((n, h2), jnp.float32),
        ),
    )(p, hw, dinv, b, w_next)


def _tc_gat_prep(p, hw, dinv, b, wa, aa, heads, hdim):
    def body(p_ref, hw_ref, dinv_ref, b_ref, wa_ref, aa_ref,
             hh0_ref, hh1_ref, hh2_ref, hh3_ref, ad_ref, exs_ref):
        dinv = dinv_ref[...]
        hcur = dinv * (p_ref[0] + p_ref[1]) + dinv * dinv * hw_ref[...]
        hcur = jnp.maximum(hcur + b_ref[...], 0.0)
        hh = jnp.dot(hcur, wa_ref[...], preferred_element_type=jnp.float32)
        hh0_ref[...] = hh[:, 0 * hdim:1 * hdim]
        hh1_ref[...] = hh[:, 1 * hdim:2 * hdim]
        hh2_ref[...] = hh[:, 2 * hdim:3 * hdim]
        hh3_ref[...] = hh[:, 3 * hdim:4 * hdim]
        ad = jnp.dot(hh, aa_ref[...], preferred_element_type=jnp.float32)
        ad_ref[...] = ad
        es = ad[:, :heads] + ad[:, heads:]
        es = jnp.where(es > 0, es, 0.2 * es)
        exs_ref[...] = jnp.exp(es)

    n = hw.shape[0]
    return pl.pallas_call(
        body,
        compiler_params=pltpu.CompilerParams(
            vmem_limit_bytes=100 * 1024 * 1024),
        out_shape=(
            jax.ShapeDtypeStruct((n, hdim), jnp.float32),
            jax.ShapeDtypeStruct((n, hdim), jnp.float32),
            jax.ShapeDtypeStruct((n, hdim), jnp.float32),
            jax.ShapeDtypeStruct((n, hdim), jnp.float32),
            jax.ShapeDtypeStruct((n, 2 * heads), jnp.float32),
            jax.ShapeDtypeStruct((n, heads), jnp.float32),
        ),
    )(p, hw, dinv, b, wa, aa)


def _tc_gat_mid(s_p, exs, hh0, hh1, hh2, hh3, heads, hdim):
    def body(sp_ref, exs_ref, hh0_ref, hh1_ref, hh2_ref, hh3_ref,
             sinv_ref, sg_ref):
        s = sp_ref[0][:, :heads] + sp_ref[1][:, :heads] + exs_ref[...]
        sinv = 1.0 / jnp.maximum(s, 1e-16)
        sinv_ref[...] = sinv
        w0 = exs_ref[...] * sinv
        sg_ref[...] = jnp.concatenate(
            [w0[:, 0:1] * hh0_ref[...], w0[:, 1:2] * hh1_ref[...],
             w0[:, 2:3] * hh2_ref[...], w0[:, 3:4] * hh3_ref[...]],
            axis=1)

    n = exs.shape[0]
    nb = 10
    bn = n // nb
    pad16 = s_p.shape[2]
    hhspec = pl.BlockSpec((bn, hdim), lambda i: (i, 0))
    return pl.pallas_call(
        body,
        grid=(nb,),
        in_specs=[
            pl.BlockSpec((2, bn, pad16), lambda i: (0, i, 0)),
            pl.BlockSpec((bn, heads), lambda i: (i, 0)),
            hhspec, hhspec, hhspec, hhspec,
        ],
        out_specs=(
            pl.BlockSpec((bn, heads), lambda i: (i, 0)),
            pl.BlockSpec((bn, heads * hdim), lambda i: (i, 0)),
        ),
        out_shape=(
            jax.ShapeDtypeStruct((n, heads), jnp.float32),
            jax.ShapeDtypeStruct((n, heads * hdim), jnp.float32),
        ),
    )(s_p, exs, hh0, hh1, hh2, hh3)


def _tc_final(gat_p, sg, ba, batch2d, wf1, bf1, wf2, bf2, ng, heads,
              hdim):
    n = sg.shape[0]
    nb = 10
    bn = n // nb
    nc = wf2.shape[1]

    def body(g_ref, sg_ref, ba_ref, b_ref, wf1_ref, bf1_ref, wf2_ref,
             bf2_ref, out_ref, summ_acc, cnt_acc):
        i = pl.program_id(0)
        parts = [g_ref[hd][0] + g_ref[hd][1] for hd in range(heads)]
        gat = jnp.concatenate(parts, axis=1) + sg_ref[...] + ba_ref[...]
        gid = jax.lax.broadcasted_iota(jnp.int32, (bn, ng), 1)
        oh = (b_ref[...] == gid).astype(jnp.float32)
        summ = lax.dot_general(oh, gat, (((0,), (0,)), ((), ())),
                               preferred_element_type=jnp.float32)
        cnt = lax.dot_general(oh, jnp.ones((bn, 1), jnp.float32),
                              (((0,), (0,)), ((), ())),
                              preferred_element_type=jnp.float32)

        @pl.when(i == 0)
        def _():
            summ_acc[...] = jnp.zeros_like(summ_acc)
            cnt_acc[...] = jnp.zeros_like(cnt_acc)

        summ_acc[...] += summ
        cnt_acc[...] += cnt

        @pl.when(i == nb - 1)
        def _():
            pooled = summ_acc[...] / jnp.maximum(cnt_acc[...], 1.0)
            o = jnp.maximum(
                jnp.dot(pooled, wf1_ref[...],
                        preferred_element_type=jnp.float32)
                + bf1_ref[...], 0.0)
            out_ref[...] = jnp.dot(
                o, wf2_ref[...], preferred_element_type=jnp.float32) \
                + bf2_ref[...]

    full = lambda *shape: pl.BlockSpec(shape, lambda i: tuple(
        0 for _ in shape))
    return pl.pallas_call(
        body,
        grid=(nb,),
        in_specs=[
            pl.BlockSpec((heads, 2, bn, hdim), lambda i: (0, 0, i, 0)),
            pl.BlockSpec((bn, heads * hdim), lambda i: (i, 0)),
            full(1, heads * hdim),
            pl.BlockSpec((bn, 1), lambda i: (i, 0)),
            full(*wf1.shape),
            full(*bf1.shape),
            full(*wf2.shape),
            full(*bf2.shape),
        ],
        out_specs=pl.BlockSpec((ng, nc), lambda i: (0, 0)),
        scratch_shapes=[
            pltpu.VMEM((ng, heads * hdim), jnp.float32),
            pltpu.VMEM((ng, 1), jnp.float32),
        ],
        out_shape=jax.ShapeDtypeStruct((ng, nc), jnp.float32),
    )(gat_p, sg, ba, batch2d, wf1, bf1, wf2, bf2)


def kernel(x, edge_index, batch, W1, b1, W2, b2, Wa, a_src, a_dst, ba,
           Wf1, bf1, Wf2, bf2):
    n, f = x.shape
    e = edge_index.shape[1]
    heads, hdim = a_src.shape
    ng = 64
    h = W1.shape[1]

    src = edge_index[0]
    dst = edge_index[1]
    batch2d = batch.reshape(n, 1)
    b1r = b1.reshape(1, h)
    b2r = b2.reshape(1, h)
    bar = ba.reshape(1, heads * hdim)
    bf1r = bf1.reshape(1, -1)
    bf2r = bf2.reshape(1, -1)
    eye = jnp.eye(heads, dtype=jnp.float32)
    asrc_m = (a_src[:, :, None] * eye[:, None, :]).reshape(heads * hdim,
                                                           heads)
    adst_m = (a_dst[:, :, None] * eye[:, None, :]).reshape(heads * hdim,
                                                           heads)
    aa = jnp.concatenate([asrc_m, adst_m], axis=1)

    deg_p = _sc_degree(n, e)(dst)
    g1, hw1, dinv = _tc_prep(x, W1, deg_p)
    p1 = _sc_edge(n, e, h)(g1, src, dst)
    g2, hw2 = _tc_layer(p1, hw1, dinv, b1r, W2)
    p2 = _sc_edge(n, e, h)(g2, src, dst)
    hh0, hh1, hh2, hh3, ad, exs = _tc_gat_prep(p2, hw2, dinv, b2r, Wa,
                                               aa, heads, hdim)
    s_p, exv = _sc_gat1(n, e, heads)(ad.reshape(-1), src, dst)
    sinv, sg = _tc_gat_mid(s_p, exs, hh0, hh1, hh2, hh3, heads, hdim)
    gat_p = _sc_gat2(n, e, heads, hdim)(hh0, hh1, hh2, hh3,
                                        sinv.reshape(-1),
                                        exv.reshape(-1), src, dst)
    return _tc_final(gat_p, sg, bar, batch2d, Wf1, bf1r, Wf2, bf2r, ng,
                     heads, hdim)


# speedup vs baseline: 29.3150x; 1.0106x over previous
import functools

import jax
import jax.numpy as jnp
from jax import lax
from jax.experimental import pallas as pl
from jax.experimental.pallas import tpu as pltpu
from jax.experimental.pallas import tpu_sc as plsc

NCORE = 2
NTILE = 16
LANES = 16
C = 128
PAD = 16
RB = 128

_SC_PARAMS = pltpu.CompilerParams(needs_layout_passes=False,
                                  use_tc_tiling_on_sc=False)


def _mesh():
    return plsc.VectorSubcoreMesh(
        core_axis_name="c", subcore_axis_name="s",
        num_cores=NCORE, num_subcores=NTILE)


def _wid():
    return lax.axis_index("s")


def _zero_rows(buf, nrows, ncols):
    zeros16 = jnp.zeros((LANES,), jnp.float32)

    def row(r, _):
        for k in range(ncols // LANES):
            buf[r, pl.ds(k * LANES, LANES)] = zeros16
        return 0

    lax.fori_loop(0, nrows, row, 0)


def _blocks(n):
    nbf = n // RB
    tail = n - nbf * RB
    assert tail % 8 == 0
    return nbf, tail


def _nblk(w, nbf):
    return nbf // NTILE + jnp.where(w < nbf % NTILE, 1, 0)


def _fill_spmem(accum, iobuf, n):
    w = _wid()
    nbf, tail = _blocks(n)
    nb = _nblk(w, nbf)

    def blk(k, _):
        base = (w + k * NTILE) * RB
        pltpu.sync_copy(iobuf, accum.at[pl.ds(base, RB)])
        return 0

    lax.fori_loop(0, nb, blk, 0)
    if tail:
        @pl.when(w == NTILE - 1)
        def _():
            pltpu.sync_copy(iobuf.at[pl.ds(0, tail)],
                            accum.at[pl.ds(nbf * RB, tail)])


def _drain_spmem(accum, iobuf, out_slice_fn, n):
    w = _wid()
    nbf, tail = _blocks(n)
    nb = _nblk(w, nbf)

    def blk(k, _):
        base = (w + k * NTILE) * RB
        pltpu.sync_copy(accum.at[pl.ds(base, RB)], iobuf)
        pltpu.sync_copy(iobuf, out_slice_fn(base, RB))
        return 0

    lax.fori_loop(0, nb, blk, 0)
    if tail:
        @pl.when(w == NTILE - 1)
        def _():
            pltpu.sync_copy(accum.at[pl.ds(nbf * RB, tail)],
                            iobuf.at[pl.ds(0, tail)])
            pltpu.sync_copy(iobuf.at[pl.ds(0, tail)],
                            out_slice_fn(nbf * RB, tail))


def _tile_chunks(per_part, w):
    return per_part // NTILE + jnp.where(w < per_part % NTILE, 1, 0)


def _sc_degree(n, e):
    per_core = (e // C) // NCORE
    mesh = _mesh()

    @functools.partial(
        pl.kernel, mesh=mesh, compiler_params=_SC_PARAMS,
        out_type=jax.ShapeDtypeStruct((NCORE, n, PAD), jnp.float32),
        scratch_types=[
            pltpu.VMEM((2, C), jnp.int32),
            pltpu.VMEM((C, PAD), jnp.float32),
            pltpu.VMEM((RB, PAD), jnp.float32),
            pltpu.VMEM_SHARED((n, PAD), jnp.float32),
        ],
    )
    def k(epk_hbm, out_hbm, ebuf, ones_buf, iobuf, accum):
        c = lax.axis_index("c")
        w = _wid()
        pat = jnp.where(lax.iota(jnp.int32, LANES) == 0, 1.0, 0.0)

        def row(r, _):
            ones_buf[r, pl.ds(0, LANES)] = pat
            return 0

        lax.fori_loop(0, C, row, 0)
        _zero_rows(iobuf, RB, PAD)
        _fill_spmem(accum, iobuf, n)
        plsc.subcore_barrier()

        def chunk(jj, _):
            ch = c * per_core + w + jj * NTILE
            pltpu.sync_copy(epk_hbm.at[ch], ebuf)
            pltpu.sync_copy(ones_buf, accum.at[ebuf.at[1]], add=True)
            return 0

        lax.fori_loop(0, _tile_chunks(per_core, w), chunk, 0)
        plsc.subcore_barrier()
        _drain_spmem(accum, iobuf,
                     lambda base, sz: out_hbm.at[c, pl.ds(base, sz)], n)

    return k


def _sc_edge(n, e, h):
    per_core = (e // C) // NCORE
    mesh = _mesh()

    @functools.partial(
        pl.kernel, mesh=mesh, compiler_params=_SC_PARAMS,
        out_type=jax.ShapeDtypeStruct((NCORE, n, h), jnp.float32),
        scratch_types=[
            pltpu.VMEM((2, 2, C), jnp.int32),
            pltpu.VMEM((2, C, h), jnp.float32),
            pltpu.VMEM((RB, h), jnp.float32),
            pltpu.VMEM_SHARED((n, h), jnp.float32),
            pltpu.SemaphoreType.DMA,
            pltpu.SemaphoreType.DMA,
            pltpu.SemaphoreType.DMA,
            pltpu.SemaphoreType.DMA,
        ],
    )
    def k(g_hbm, epk_hbm, out_hbm, eidx, rows, iobuf, accum, sem0, sem1,
          ssem0, ssem1):
        c = lax.axis_index("c")
        w = _wid()
        _zero_rows(iobuf, RB, h)
        _fill_spmem(accum, iobuf, n)
        plsc.subcore_barrier()

        nj = _tile_chunks(per_core, w)
        sems = (sem0, sem1)
        ssems = (ssem0, ssem1)

        def issue(jj, par):
            ch = c * per_core + w + jj * NTILE

            @pl.when(jj >= 2)
            def _():
                pltpu.make_async_copy(
                    rows.at[par], accum.at[eidx.at[par, 1]],
                    ssems[par]).wait()

            pltpu.sync_copy(epk_hbm.at[ch], eidx.at[par])
            pltpu.async_copy(g_hbm.at[eidx.at[par, 0]], rows.at[par],
                             sems[par])

        def process(jj, par):
            pltpu.make_async_copy(g_hbm.at[eidx.at[par, 0]],
                                  rows.at[par], sems[par]).wait()
            pltpu.async_copy(rows.at[par], accum.at[eidx.at[par, 1]],
                             ssems[par], add=True)

        @pl.when(nj > 0)
        def _():
            issue(0, 0)

        def body(jj, _):
            for par in (0, 1):
                @pl.when(jj % 2 == par)
                def _():
                    @pl.when(jj + 1 < nj)
                    def _():
                        issue(jj + 1, 1 - par)

                    process(jj, par)
            return 0

        lax.fori_loop(0, nj, body, 0)
        for par in (0, 1):
            @pl.when(nj >= 1 + par)
            def _():
                pltpu.make_async_copy(
                    rows.at[par], accum.at[eidx.at[par, 1]],
                    ssems[par]).wait()

        plsc.subcore_barrier()
        _drain_spmem(accum, iobuf,
                     lambda base, sz: out_hbm.at[c, pl.ds(base, sz)], n)

    return k


def _sc_gat1(n, e, heads):
    per_core = (e // C) // NCORE
    mesh = _mesh()
    ncols = 2 * heads

    @functools.partial(
        pl.kernel, mesh=mesh, compiler_params=_SC_PARAMS,
        out_type=(
            jax.ShapeDtypeStruct((NCORE, n, PAD), jnp.float32),
            jax.ShapeDtypeStruct((e, PAD), jnp.float32),
        ),
        scratch_types=[
            pltpu.VMEM((n * ncols,), jnp.float32),
            pltpu.VMEM((2, 2, C), jnp.int32),
            pltpu.VMEM((2, C, PAD), jnp.float32),
            pltpu.VMEM((RB, PAD), jnp.float32),
            pltpu.VMEM_SHARED((n, PAD), jnp.float32),
            pltpu.SemaphoreType.DMA,
            pltpu.SemaphoreType.DMA,
            pltpu.SemaphoreType.DMA,
            pltpu.SemaphoreType.DMA,
        ],
    )
    def k(ad_hbm, epk_hbm, s_out, ex_out, ad_tab, eidx, exbuf, iobuf,
          accum, ssem0, ssem1, osem0, osem1):
        c = lax.axis_index("c")
        w = _wid()
        pltpu.sync_copy(ad_hbm, ad_tab)
        _zero_rows(iobuf, RB, PAD)
        _fill_spmem(accum, iobuf, n)
        plsc.subcore_barrier()

        iota16 = lax.iota(jnp.int32, LANES)
        lane_ok = iota16 < heads
        maxi = n * ncols - 1
        nj = _tile_chunks(per_core, w)
        ssems = (ssem0, ssem1)
        osems = (osem0, osem1)

        def issue(jj, par):
            ch = c * per_core + w + jj * NTILE
            pltpu.sync_copy(epk_hbm.at[ch], eidx.at[par])

        def drain(jj, par, ebase):
            pltpu.make_async_copy(exbuf.at[par],
                                  accum.at[eidx.at[par, 1]],
                                  ssems[par]).wait()
            pltpu.make_async_copy(exbuf.at[par],
                                  ex_out.at[pl.ds(ebase, C)],
                                  osems[par]).wait()

        def process(jj, par):
            ch = c * per_core + w + jj * NTILE
            ebase = ch * C

            @pl.when(jj >= 2)
            def _():
                drain(jj, par, ebase)

            for g in range(C // LANES):
                sv = eidx[par, 0, pl.ds(g * LANES, LANES)]
                dv = eidx[par, 1, pl.ds(g * LANES, LANES)]
                for j in range(LANES):
                    si = sv[j] * ncols + iota16
                    di = dv[j] * ncols + heads + iota16
                    a = plsc.load_gather(ad_tab, [jnp.minimum(si, maxi)])
                    b = plsc.load_gather(ad_tab, [jnp.minimum(di, maxi)])
                    ee = a + b
                    ee = jnp.where(ee > 0, ee, 0.2 * ee)
                    ex = jnp.where(lane_ok, jnp.exp(ee), 0.0)
                    exbuf[par, g * LANES + j, pl.ds(0, LANES)] = ex
            pltpu.async_copy(exbuf.at[par], accum.at[eidx.at[par, 1]],
                             ssems[par], add=True)
            pltpu.async_copy(exbuf.at[par], ex_out.at[pl.ds(ebase, C)],
                             osems[par])

        @pl.when(nj > 0)
        def _():
            issue(0, 0)

        def body(jj, _):
            for par in (0, 1):
                @pl.when(jj % 2 == par)
                def _():
                    @pl.when(jj + 1 < nj)
                    def _():
                        issue(jj + 1, 1 - par)

                    process(jj, par)
            return 0

        lax.fori_loop(0, nj, body, 0)
        for par in (0, 1):
            @pl.when(nj >= 1 + par)
            def _():
                pltpu.make_async_copy(exbuf.at[par],
                                      accum.at[eidx.at[par, 1]],
                                      ssems[par]).wait()
                pltpu.make_async_copy(exbuf.at[par],
                                      ex_out.at[pl.ds(0, C)],
                                      osems[par]).wait()

        plsc.subcore_barrier()
        _drain_spmem(accum, iobuf,
                     lambda base, sz: s_out.at[c, pl.ds(base, sz)], n)

    return k


def _sc_gat2(n, e, heads, hdim):
    per_core = (e // C) // NCORE
    mesh = _mesh()

    @functools.partial(
        pl.kernel, mesh=mesh, compiler_params=_SC_PARAMS,
        out_type=jax.ShapeDtypeStruct((heads, NCORE, n, hdim),
                                      jnp.float32),
        scratch_types=[
            pltpu.VMEM((n * heads,), jnp.float32),
            pltpu.VMEM((2, 2, C), jnp.int32),
            pltpu.VMEM((2, C * PAD), jnp.float32),
            pltpu.VMEM((2, C, hdim), jnp.float32),
            pltpu.VMEM((C,), jnp.float32),
            pltpu.VMEM((RB, hdim), jnp.float32),
            pltpu.VMEM_SHARED((n, hdim), jnp.float32),
            pltpu.SemaphoreType.DMA,
            pltpu.SemaphoreType.DMA,
            pltpu.SemaphoreType.DMA,
            pltpu.SemaphoreType.DMA,
        ],
    )
    def k(hh0_hbm, hh1_hbm, hh2_hbm, hh3_hbm, sinv_hbm, ex_hbm,
          epk_hbm, out_hbm, sinv_tab, eidx, exch, rows, albuf, iobuf,
          accum, sem0, sem1, ssem0, ssem1):
        c = lax.axis_index("c")
        w = _wid()
        pltpu.sync_copy(sinv_hbm, sinv_tab)
        iota16 = lax.iota(jnp.int32, LANES)
        nj = _tile_chunks(per_core, w)
        sems = (sem0, sem1)
        ssems = (ssem0, ssem1)

        for hd, hh_hbm in enumerate([hh0_hbm, hh1_hbm, hh2_hbm, hh3_hbm]):
            plsc.subcore_barrier()
            _zero_rows(iobuf, RB, hdim)
            _fill_spmem(accum, iobuf, n)
            plsc.subcore_barrier()

            def issue(jj, par, hh_hbm=hh_hbm):
                ch = c * per_core + w + jj * NTILE

                @pl.when(jj >= 2)
                def _():
                    pltpu.make_async_copy(
                        rows.at[par], accum.at[eidx.at[par, 1]],
                        ssems[par]).wait()

                pltpu.sync_copy(epk_hbm.at[ch], eidx.at[par])
                pltpu.sync_copy(ex_hbm.at[pl.ds(ch * C * PAD, C * PAD)],
                                exch.at[par])
                pltpu.async_copy(hh_hbm.at[eidx.at[par, 0]],
                                 rows.at[par], sems[par])

            def process(jj, par, hd=hd, hh_hbm=hh_hbm):
                pltpu.make_async_copy(hh_hbm.at[eidx.at[par, 0]],
                                      rows.at[par], sems[par]).wait()
                for g in range(C // LANES):
                    dv = eidx[par, 1, pl.ds(g * LANES, LANES)]
                    ei = iota16 * PAD + g * LANES * PAD + hd
                    exv = plsc.load_gather(exch.at[par], [ei])
                    siv = plsc.load_gather(sinv_tab,
                                           [dv * heads + hd])
                    albuf[pl.ds(g * LANES, LANES)] = exv * siv

                def scale(g2, _):
                    va = albuf[pl.ds(g2 * LANES, LANES)]
                    for j in range(LANES):
                        i = g2 * LANES + j
                        aA = va[j]
                        for kk in range(hdim // LANES):
                            v = rows[par, i, pl.ds(kk * LANES, LANES)]
                            rows[par, i, pl.ds(kk * LANES, LANES)] = \
                                v * aA
                    return 0

                lax.fori_loop(0, C // LANES, scale, 0)
                pltpu.async_copy(rows.at[par], accum.at[eidx.at[par, 1]],
                                 ssems[par], add=True)

            @pl.when(nj > 0)
            def _():
                issue(0, 0)

            def body(jj, _):
                for par in (0, 1):
                    @pl.when(jj % 2 == par)
                    def _():
                        @pl.when(jj + 1 < nj)
                        def _():
                            issue(jj + 1, 1 - par)

                        process(jj, par)
                return 0

            lax.fori_loop(0, nj, body, 0)
            for par in (0, 1):
                @pl.when(nj >= 1 + par)
                def _():
                    pltpu.make_async_copy(
                        rows.at[par], accum.at[eidx.at[par, 1]],
                        ssems[par]).wait()

            plsc.subcore_barrier()
            _drain_spmem(
                accum, iobuf,
                lambda base, sz: out_hbm.at[hd, c, pl.ds(base, sz)], n)

    return k


def _tc_prep(x, w1, deg_p):
    def body(x_ref, w1_ref, degp_ref, g_ref, hw_ref, dinv_ref):
        deg = degp_ref[0][:, 0:1] + degp_ref[1][:, 0:1] + 1.0
        dinv = lax.rsqrt(deg)
        hw = jnp.dot(x_ref[...], w1_ref[...],
                     preferred_element_type=jnp.float32)
        hw_ref[...] = hw
        g_ref[...] = dinv * hw
        dinv_ref[...] = dinv

    n = x.shape[0]
    h = w1.shape[1]
    return pl.pallas_call(
        body,
        compiler_params=pltpu.CompilerParams(
            vmem_limit_bytes=100 * 1024 * 1024),
        out_shape=(
            jax.ShapeDtypeStruct((n, h), jnp.float32),
            jax.ShapeDtypeStruct((n, h), jnp.float32),
            jax.ShapeDtypeStruct((n, 1), jnp.float32),
        ),
    )(x, w1, deg_p)


def _tc_layer(p, hw, dinv, b, w_next):
    def body(p_ref, hw_ref, dinv_ref, b_ref, w_ref, g_ref, hw2_ref):
        dinv = dinv_ref[...]
        hcur = dinv * (p_ref[0] + p_ref[1]) + dinv * dinv * hw_ref[...]
        hcur = jnp.maximum(hcur + b_ref[...], 0.0)
        hw2 = jnp.dot(hcur, w_ref[...], preferred_element_type=jnp.float32)
        hw2_ref[...] = hw2
        g_ref[...] = dinv * hw2

    n = hw.shape[0]
    h2 = w_next.shape[1]
    return pl.pallas_call(
        body,
        compiler_params=pltpu.CompilerParams(
            vmem_limit_bytes=100 * 1024 * 1024),
        out_shape=(
            jax.ShapeDtypeStruct((n, h2), jnp.float32),
            jax.ShapeDtypeStruct((n, h2), jnp.float32),
        ),
    )(p, hw, dinv, b, w_next)


def _tc_gat_prep(p, hw, dinv, b, wa, aa, heads, hdim):
    def body(p_ref, hw_ref, dinv_ref, b_ref, wa_ref, aa_ref,
             hh0_ref, hh1_ref, hh2_ref, hh3_ref, ad_ref, exs_ref):
        dinv = dinv_ref[...]
        hcur = dinv * (p_ref[0] + p_ref[1]) + dinv * dinv * hw_ref[...]
        hcur = jnp.maximum(hcur + b_ref[...], 0.0)
        hh = jnp.dot(hcur, wa_ref[...], preferred_element_type=jnp.float32)
        hh0_ref[...] = hh[:, 0 * hdim:1 * hdim]
        hh1_ref[...] = hh[:, 1 * hdim:2 * hdim]
        hh2_ref[...] = hh[:, 2 * hdim:3 * hdim]
        hh3_ref[...] = hh[:, 3 * hdim:4 * hdim]
        ad = jnp.dot(hh, aa_ref[...], preferred_element_type=jnp.float32)
        ad_ref[...] = ad
        es = ad[:, :heads] + ad[:, heads:]
        es = jnp.where(es > 0, es, 0.2 * es)
        exs_ref[...] = jnp.exp(es)

    n = hw.shape[0]
    return pl.pallas_call(
        body,
        compiler_params=pltpu.CompilerParams(
            vmem_limit_bytes=100 * 1024 * 1024),
        out_shape=(
            jax.ShapeDtypeStruct((n, hdim), jnp.float32),
            jax.ShapeDtypeStruct((n, hdim), jnp.float32),
            jax.ShapeDtypeStruct((n, hdim), jnp.float32),
            jax.ShapeDtypeStruct((n, hdim), jnp.float32),
            jax.ShapeDtypeStruct((n, 2 * heads), jnp.float32),
            jax.ShapeDtypeStruct((n, heads), jnp.float32),
        ),
    )(p, hw, dinv, b, wa, aa)


def _tc_gat_mid(s_p, exs, hh0, hh1, hh2, hh3, heads, hdim):
    def body(sp_ref, exs_ref, hh0_ref, hh1_ref, hh2_ref, hh3_ref,
             sinv_ref, sg_ref):
        s = sp_ref[0][:, :heads] + sp_ref[1][:, :heads] + exs_ref[...]
        sinv = 1.0 / jnp.maximum(s, 1e-16)
        sinv_ref[...] = sinv
        w0 = exs_ref[...] * sinv
        sg_ref[...] = jnp.concatenate(
            [w0[:, 0:1] * hh0_ref[...], w0[:, 1:2] * hh1_ref[...],
             w0[:, 2:3] * hh2_ref[...], w0[:, 3:4] * hh3_ref[...]],
            axis=1)

    n = exs.shape[0]
    nb = 10
    bn = n // nb
    pad16 = s_p.shape[2]
    hhspec = pl.BlockSpec((bn, hdim), lambda i: (i, 0))
    return pl.pallas_call(
        body,
        grid=(nb,),
        in_specs=[
            pl.BlockSpec((2, bn, pad16), lambda i: (0, i, 0)),
            pl.BlockSpec((bn, heads), lambda i: (i, 0)),
            hhspec, hhspec, hhspec, hhspec,
        ],
        out_specs=(
            pl.BlockSpec((bn, heads), lambda i: (i, 0)),
            pl.BlockSpec((bn, heads * hdim), lambda i: (i, 0)),
        ),
        out_shape=(
            jax.ShapeDtypeStruct((n, heads), jnp.float32),
            jax.ShapeDtypeStruct((n, heads * hdim), jnp.float32),
        ),
    )(s_p, exs, hh0, hh1, hh2, hh3)


def _tc_final(gat_p, sg, ba, batch2d, wf1, bf1, wf2, bf2, ng, heads,
              hdim):
    n = sg.shape[0]
    nb = 10
    bn = n // nb
    nc = wf2.shape[1]

    def body(g_ref, sg_ref, ba_ref, b_ref, wf1_ref, bf1_ref, wf2_ref,
             bf2_ref, out_ref, summ_acc, cnt_acc):
        i = pl.program_id(0)
        parts = [g_ref[hd][0] + g_ref[hd][1] for hd in range(heads)]
        gat = jnp.concatenate(parts, axis=1) + sg_ref[...] + ba_ref[...]
        gid = jax.lax.broadcasted_iota(jnp.int32, (bn, ng), 1)
        oh = (b_ref[...] == gid).astype(jnp.float32)
        summ = lax.dot_general(oh, gat, (((0,), (0,)), ((), ())),
                               preferred_element_type=jnp.float32)
        cnt = lax.dot_general(oh, jnp.ones((bn, 1), jnp.float32),
                              (((0,), (0,)), ((), ())),
                              preferred_element_type=jnp.float32)

        @pl.when(i == 0)
        def _():
            summ_acc[...] = jnp.zeros_like(summ_acc)
            cnt_acc[...] = jnp.zeros_like(cnt_acc)

        summ_acc[...] += summ
        cnt_acc[...] += cnt

        @pl.when(i == nb - 1)
        def _():
            pooled = summ_acc[...] / jnp.maximum(cnt_acc[...], 1.0)
            o = jnp.maximum(
                jnp.dot(pooled, wf1_ref[...],
                        preferred_element_type=jnp.float32)
                + bf1_ref[...], 0.0)
            out_ref[...] = jnp.dot(
                o, wf2_ref[...], preferred_element_type=jnp.float32) \
                + bf2_ref[...]

    full = lambda *shape: pl.BlockSpec(shape, lambda i: tuple(
        0 for _ in shape))
    return pl.pallas_call(
        body,
        grid=(nb,),
        in_specs=[
            pl.BlockSpec((heads, 2, bn, hdim), lambda i: (0, 0, i, 0)),
            pl.BlockSpec((bn, heads * hdim), lambda i: (i, 0)),
            full(1, heads * hdim),
            pl.BlockSpec((bn, 1), lambda i: (i, 0)),
            full(*wf1.shape),
            full(*bf1.shape),
            full(*wf2.shape),
            full(*bf2.shape),
        ],
        out_specs=pl.BlockSpec((ng, nc), lambda i: (0, 0)),
        scratch_shapes=[
            pltpu.VMEM((ng, heads * hdim), jnp.float32),
            pltpu.VMEM((ng, 1), jnp.float32),
        ],
        out_shape=jax.ShapeDtypeStruct((ng, nc), jnp.float32),
    )(gat_p, sg, ba, batch2d, wf1, bf1, wf2, bf2)


def kernel(x, edge_index, batch, W1, b1, W2, b2, Wa, a_src, a_dst, ba,
           Wf1, bf1, Wf2, bf2):
    n, f = x.shape
    e = edge_index.shape[1]
    heads, hdim = a_src.shape
    ng = 64
    h = W1.shape[1]

    edges_pk = edge_index.reshape(2, e // C, C).transpose(1, 0, 2)
    batch2d = batch.reshape(n, 1)
    b1r = b1.reshape(1, h)
    b2r = b2.reshape(1, h)
    bar = ba.reshape(1, heads * hdim)
    bf1r = bf1.reshape(1, -1)
    bf2r = bf2.reshape(1, -1)
    eye = jnp.eye(heads, dtype=jnp.float32)
    asrc_m = (a_src[:, :, None] * eye[:, None, :]).reshape(heads * hdim,
                                                           heads)
    adst_m = (a_dst[:, :, None] * eye[:, None, :]).reshape(heads * hdim,
                                                           heads)
    aa = jnp.concatenate([asrc_m, adst_m], axis=1)

    deg_p = _sc_degree(n, e)(edges_pk)
    g1, hw1, dinv = _tc_prep(x, W1, deg_p)
    p1 = _sc_edge(n, e, h)(g1, edges_pk)
    g2, hw2 = _tc_layer(p1, hw1, dinv, b1r, W2)
    p2 = _sc_edge(n, e, h)(g2, edges_pk)
    hh0, hh1, hh2, hh3, ad, exs = _tc_gat_prep(p2, hw2, dinv, b2r, Wa,
                                               aa, heads, hdim)
    s_p, exv = _sc_gat1(n, e, heads)(ad.reshape(-1), edges_pk)
    sinv, sg = _tc_gat_mid(s_p, exs, hh0, hh1, hh2, hh3, heads, hdim)
    gat_p = _sc_gat2(n, e, heads, hdim)(hh0, hh1, hh2, hh3,
                                        sinv.reshape(-1),
                                        exv.reshape(-1), edges_pk)
    return _tc_final(gat_p, sg, bar, batch2d, Wf1, bf1r, Wf2, bf2r, ng,
                     heads, hdim)
